# baseline (speedup 1.0000x reference)
import functools

import jax
import jax.numpy as jnp
from jax import lax
from jax.experimental import pallas as pl
from jax.experimental.pallas import tpu as pltpu
from jax.experimental.pallas import tpu_sc as plsc

N = 10000
E = 320000
D = 128
ED = 16
G = 64

NC = 2
NS = 16
NW = NC * NS
CH = 64
NCH = 5120
EP = NCH * CH
CN0 = 160
CN1 = 160
NP_ = 10240
ROWS_PER_TILE = NP_ // NS


def _leaky(v):
    return jnp.where(v > 0, v, v * jnp.float32(0.01))


_mesh = plsc.VectorSubcoreMesh(
    core_axis_name="c", subcore_axis_name="s", num_cores=NC, num_subcores=NS
)


@functools.partial(
    pl.kernel,
    out_type=jax.ShapeDtypeStruct((NC * NP_, D), jnp.float32),
    mesh=_mesh,
    scratch_types=[
        pltpu.VMEM((4, 2 * CH), jnp.int32),
        pltpu.VMEM((4, CH), jnp.int32),
        pltpu.VMEM((CH, D), jnp.float32),
        pltpu.VMEM((CH, D), jnp.float32),
        pltpu.VMEM((CH, D), jnp.float32),
        pltpu.VMEM((CH, D), jnp.float32),
        pltpu.VMEM_SHARED((NP_, D), jnp.float32),
        pltpu.SemaphoreType.DMA,
        pltpu.SemaphoreType.DMA,
        pltpu.SemaphoreType.DMA,
        pltpu.SemaphoreType.DMA,
        pltpu.SemaphoreType.DMA,
        pltpu.SemaphoreType.DMA,
        pltpu.SemaphoreType.DMA,
        pltpu.SemaphoreType.DMA,
        pltpu.SemaphoreType.DMA,
        pltpu.SemaphoreType.DMA,
    ],
)
def _sc_agg(x_hbm, sd_hbm, e_hbm, out_hbm, isd, idst,
            rows0, rows1, evs0, evs1, agg,
            si0, si1, si2, si3, sg0, sg1, se0, se1, ss0, ss1):
    c = lax.axis_index("c")
    s = lax.axis_index("s")
    nj = jnp.where(c == 0, CN0, CN1)
    base = jnp.where(c == 0, s * CN0, NS * CN0 + s * CN1)
    row0 = s * ROWS_PER_TILE

    isems = (si0, si1, si2, si3)
    rbufs = (rows0, rows1)
    ebufs = (evs0, evs1)
    gsems = (sg0, sg1)
    esems = (se0, se1)
    ssems = (ss0, ss1)

    def zero_row(i, carry):
        for k in range(D // 16):
            rows0[i, pl.ds(k * 16, 16)] = jnp.zeros((16,), jnp.float32)
        return carry

    lax.fori_loop(0, CH, zero_row, 0)
    for k in range(ROWS_PER_TILE // CH):
        pltpu.sync_copy(rows0.at[pl.ds(0, CH)],
                        agg.at[pl.ds(row0 + k * CH, CH)])
    plsc.subcore_barrier()

    def issue_idx(j, slot):
        g = base + j
        pltpu.async_copy(sd_hbm.at[pl.ds(g * (2 * CH), 2 * CH)],
                         isd.at[slot], isems[slot])

    def wait_idx(j, slot):
        g = base + j
        pltpu.make_async_copy(sd_hbm.at[pl.ds(g * (2 * CH), 2 * CH)],
                              isd.at[slot], isems[slot]).wait()
        for k in range(CH // 16):
            idst[slot, pl.ds(k * 16, 16)] = isd[slot, pl.ds(CH + k * 16, 16)]

    def issue_data(j, slot, b):
        g = base + j
        pltpu.async_copy(x_hbm.at[isd.at[slot, pl.ds(0, CH)]], rbufs[b],
                         gsems[b])
        pltpu.async_copy(e_hbm.at[pl.ds(g * CH, CH)], ebufs[b], esems[b])

    def wait_data(j, slot, b):
        g = base + j
        pltpu.make_async_copy(x_hbm.at[isd.at[slot, pl.ds(0, CH)]], rbufs[b],
                              gsems[b]).wait()
        pltpu.make_async_copy(e_hbm.at[pl.ds(g * CH, CH)], ebufs[b],
                              esems[b]).wait()

    def compute(b):
        rbuf = rbufs[b]
        ebuf = ebufs[b]

        def relu_row(r, inner):
            for k in range(D // 16):
                sl = pl.ds(k * 16, 16)
                rbuf[r, sl] = jnp.maximum(rbuf[r, sl] + ebuf[r, sl],
                                          jnp.float32(0.0))
            return inner

        lax.fori_loop(0, CH, relu_row, 0)

    def scatter(slot, b):
        pltpu.async_copy(rbufs[b], agg.at[idst.at[slot]], ssems[b],
                         add=True).wait()

    for sl in range(4):
        issue_idx(jnp.int32(sl), sl)
    wait_idx(jnp.int32(0), 0)
    issue_data(jnp.int32(0), 0, 0)

    def quad_body(i, carry):
        j = 4 * i
        wait_idx(j + 1, 1)
        issue_data(j + 1, 1, 1)
        wait_data(j, 0, 0)
        compute(0)
        scatter(0, 0)

        @pl.when(j + 4 < nj)
        def _():
            issue_idx(j + 4, 0)

        wait_idx(j + 2, 2)
        issue_data(j + 2, 2, 0)
        wait_data(j + 1, 1, 1)
        compute(1)
        scatter(1, 1)

        @pl.when(j + 5 < nj)
        def _():
            issue_idx(j + 5, 1)

        wait_idx(j + 3, 3)
        issue_data(j + 3, 3, 1)
        wait_data(j + 2, 2, 0)
        compute(0)
        scatter(2, 0)

        @pl.when(j + 6 < nj)
        def _():
            issue_idx(j + 6, 2)

        @pl.when(j + 4 < nj)
        def _():
            wait_idx(j + 4, 0)
            issue_data(j + 4, 0, 0)

        wait_data(j + 3, 3, 1)
        compute(1)
        scatter(3, 1)

        @pl.when(j + 7 < nj)
        def _():
            issue_idx(j + 7, 3)

        return carry

    lax.fori_loop(0, nj // 4, quad_body, 0)
    plsc.subcore_barrier()

    for k in range(ROWS_PER_TILE // CH):
        r = row0 + k * CH
        pltpu.sync_copy(agg.at[pl.ds(r, CH)], rows0.at[pl.ds(0, CH)])
        pltpu.sync_copy(rows0.at[pl.ds(0, CH)],
                        out_hbm.at[pl.ds(c * NP_ + r, CH)])


_EB = 2000


def _edge_mlp_body(ea, We1, be1, We2, be2, e1, e2):
    a = ea[...]
    e1[...] = jnp.dot(a, We1[...], preferred_element_type=jnp.float32) + be1[...]
    e2[...] = jnp.dot(a, We2[...], preferred_element_type=jnp.float32) + be2[...]


def _edge_mlp(ea, We1, be1, We2, be2):
    grid = (E // _EB,)
    return pl.pallas_call(
        _edge_mlp_body,
        grid=grid,
        in_specs=[
            pl.BlockSpec((_EB, ED), lambda i: (i, 0)),
            pl.BlockSpec((ED, D), lambda i: (0, 0)),
            pl.BlockSpec((1, D), lambda i: (0, 0)),
            pl.BlockSpec((ED, D), lambda i: (0, 0)),
            pl.BlockSpec((1, D), lambda i: (0, 0)),
        ],
        out_specs=[
            pl.BlockSpec((_EB, D), lambda i: (i, 0)),
            pl.BlockSpec((_EB, D), lambda i: (i, 0)),
        ],
        out_shape=[
            jax.ShapeDtypeStruct((EP, D), jnp.float32),
            jax.ShapeDtypeStruct((EP, D), jnp.float32),
        ],
    )(ea, We1, be1, We2, be2)


_NB = 1000


def _node_mlp1_body(x, a0, a1, W1a, b1a, W1b, b1b, h1):
    t = x[...] + a0[...] + a1[...]
    u = _leaky(jnp.dot(t, W1a[...], preferred_element_type=jnp.float32) + b1a[...])
    h = _leaky(jnp.dot(u, W1b[...], preferred_element_type=jnp.float32) + b1b[...])
    h1[...] = jnp.concatenate([h, jnp.zeros((_NB, 64), jnp.float32)], axis=1)


def _node_mlp1(x, a0, a1, W1a, b1a, W1b, b1b):
    grid = (N // _NB,)
    full_spec = pl.BlockSpec((_NB, D), lambda i: (i, 0))
    return pl.pallas_call(
        _node_mlp1_body,
        grid=grid,
        in_specs=[
            full_spec, full_spec, full_spec,
            pl.BlockSpec((D, 32), lambda i: (0, 0)),
            pl.BlockSpec((1, 32), lambda i: (0, 0)),
            pl.BlockSpec((32, 64), lambda i: (0, 0)),
            pl.BlockSpec((1, 64), lambda i: (0, 0)),
        ],
        out_specs=pl.BlockSpec((_NB, D), lambda i: (i, 0)),
        out_shape=jax.ShapeDtypeStruct((N, D), jnp.float32),
    )(x, a0, a1, W1a, b1a, W1b, b1b)


def _node_mlp2_pool_body(h1, a0, a1, W2a, b2a, W2b, b2b, batch, g):
    i = pl.program_id(0)
    t = h1[...][:, :64] + a0[...][:, :64] + a1[...][:, :64]
    u = _leaky(jnp.dot(t, W2a[...], preferred_element_type=jnp.float32) + b2a[...])
    h2 = _leaky(jnp.dot(u, W2b[...], preferred_element_type=jnp.float32) + b2b[...])
    onehot = (batch[...] == lax.broadcasted_iota(jnp.int32, (1, G), 1)
              ).astype(jnp.float32)
    contrib = lax.dot_general(onehot, h2, (((0,), (0,)), ((), ())),
                              preferred_element_type=jnp.float32)

    @pl.when(i == 0)
    def _():
        g[...] = contrib

    @pl.when(i > 0)
    def _():
        g[...] = g[...] + contrib


def _node_mlp2_pool(h1, a0, a1, W2a, b2a, W2b, b2b, batch2d):
    grid = (N // _NB,)
    full_spec = pl.BlockSpec((_NB, D), lambda i: (i, 0))
    return pl.pallas_call(
        _node_mlp2_pool_body,
        grid=grid,
        in_specs=[
            full_spec, full_spec, full_spec,
            pl.BlockSpec((64, 128), lambda i: (0, 0)),
            pl.BlockSpec((1, 128), lambda i: (0, 0)),
            pl.BlockSpec((128, 256), lambda i: (0, 0)),
            pl.BlockSpec((1, 256), lambda i: (0, 0)),
            pl.BlockSpec((_NB, 1), lambda i: (i, 0)),
        ],
        out_specs=pl.BlockSpec((G, 256), lambda i: (0, 0)),
        out_shape=jax.ShapeDtypeStruct((G, 256), jnp.float32),
    )(h1, a0, a1, W2a, b2a, W2b, b2b, batch2d)


def _head_body(g, Wf0, bf0, Wf1, bf1, Wf2, bf2, Wr, br, out):
    t = _leaky(jnp.dot(g[...], Wf0[...], preferred_element_type=jnp.float32) + bf0[...])
    t = _leaky(jnp.dot(t, Wf1[...], preferred_element_type=jnp.float32) + bf1[...])
    t = _leaky(jnp.dot(t, Wf2[...], preferred_element_type=jnp.float32) + bf2[...])
    out[...] = jnp.dot(t, Wr[...], preferred_element_type=jnp.float32) + br[...]


def _head(g, Wf0, bf0, Wf1, bf1, Wf2, bf2, Wr, br):
    return pl.pallas_call(
        _head_body,
        out_shape=jax.ShapeDtypeStruct((G, 1), jnp.float32),
    )(g, Wf0, bf0, Wf1, bf1, Wf2, bf2, Wr, br)


def kernel(x, edge_index, edge_attr, batch, We1, be1, W1a, b1a, W1b, b1b,
           We2, be2, W2a, b2a, W2b, b2b, Wf0, bf0, Wf1, bf1, Wf2, bf2, Wr, br):
    pad = EP - E
    srcf = jnp.concatenate([edge_index[0],
                            jnp.zeros((pad,), edge_index.dtype)])
    dstf = jnp.concatenate([edge_index[1],
                            jnp.full((pad,), NP_ - 1, edge_index.dtype)])
    sd = jnp.concatenate([srcf.reshape(NCH, CH),
                          dstf.reshape(NCH, CH)], axis=1).reshape(-1)
    batch2d = batch.reshape(N, 1)
    We2p = jnp.pad(We2, ((0, 0), (0, 64)))
    be2p = jnp.pad(be2, (0, 64)).reshape(1, D)

    e1, e2 = _edge_mlp(edge_attr, We1, be1.reshape(1, D), We2p, be2p)

    p1 = _sc_agg(x, sd, e1)
    h1p = _node_mlp1(x, p1[:N], p1[NP_:NP_ + N], W1a, b1a.reshape(1, 32),
                     W1b, b1b.reshape(1, 64))

    p2 = _sc_agg(h1p, sd, e2)
    g = _node_mlp2_pool(h1p, p2[:N], p2[NP_:NP_ + N], W2a, b2a.reshape(1, 128),
                        W2b, b2b.reshape(1, 256), batch2d)

    out = _head(g, Wf0, bf0.reshape(1, 128), Wf1, bf1.reshape(1, 64),
                Wf2, bf2.reshape(1, 32), Wr, br.reshape(1, 1))
    return out.reshape(G)


# speedup vs baseline: 1.0887x; 1.0887x over previous
import functools

import jax
import jax.numpy as jnp
from jax import lax
from jax.experimental import pallas as pl
from jax.experimental.pallas import tpu as pltpu
from jax.experimental.pallas import tpu_sc as plsc

N = 10000
E = 320000
D = 128
ED = 16
G = 64

NC = 2
NS = 16
NW = NC * NS
CH = 64
NCH = 5120
EP = NCH * CH
CN0 = 224
CN1 = 96
NP_ = 10240
ROWS_PER_TILE = NP_ // NS


def _leaky(v):
    return jnp.where(v > 0, v, v * jnp.float32(0.01))


_mesh = plsc.VectorSubcoreMesh(
    core_axis_name="c", subcore_axis_name="s", num_cores=NC, num_subcores=NS
)


@functools.partial(
    pl.kernel,
    out_type=jax.ShapeDtypeStruct((NC * NP_, D), jnp.float32),
    mesh=_mesh,
    scratch_types=[
        pltpu.VMEM((4, 2 * CH), jnp.int32),
        pltpu.VMEM((4, CH), jnp.int32),
        pltpu.VMEM((CH, D), jnp.float32),
        pltpu.VMEM((CH, D), jnp.float32),
        pltpu.VMEM((CH, D), jnp.float32),
        pltpu.VMEM((CH, D), jnp.float32),
        pltpu.VMEM_SHARED((NP_, D), jnp.float32),
        pltpu.SemaphoreType.DMA,
        pltpu.SemaphoreType.DMA,
        pltpu.SemaphoreType.DMA,
        pltpu.SemaphoreType.DMA,
        pltpu.SemaphoreType.DMA,
        pltpu.SemaphoreType.DMA,
        pltpu.SemaphoreType.DMA,
        pltpu.SemaphoreType.DMA,
        pltpu.SemaphoreType.DMA,
        pltpu.SemaphoreType.DMA,
    ],
)
def _sc_agg(x_hbm, sd_hbm, e_hbm, out_hbm, isd, idst,
            rows0, rows1, evs0, evs1, agg,
            si0, si1, si2, si3, sg0, sg1, se0, se1, ss0, ss1):
    c = lax.axis_index("c")
    s = lax.axis_index("s")
    nj = jnp.where(c == 0, CN0, CN1)
    base = jnp.where(c == 0, s * CN0, NS * CN0 + s * CN1)
    row0 = s * ROWS_PER_TILE

    isems = (si0, si1, si2, si3)
    rbufs = (rows0, rows1)
    ebufs = (evs0, evs1)
    gsems = (sg0, sg1)
    esems = (se0, se1)
    ssems = (ss0, ss1)

    def zero_row(i, carry):
        for k in range(D // 16):
            rows0[i, pl.ds(k * 16, 16)] = jnp.zeros((16,), jnp.float32)
        return carry

    lax.fori_loop(0, CH, zero_row, 0)
    for k in range(ROWS_PER_TILE // CH):
        pltpu.sync_copy(rows0.at[pl.ds(0, CH)],
                        agg.at[pl.ds(row0 + k * CH, CH)])
    plsc.subcore_barrier()

    def issue_idx(j, slot):
        g = base + j
        pltpu.async_copy(sd_hbm.at[pl.ds(g * (2 * CH), 2 * CH)],
                         isd.at[slot], isems[slot])

    def wait_idx(j, slot):
        g = base + j
        pltpu.make_async_copy(sd_hbm.at[pl.ds(g * (2 * CH), 2 * CH)],
                              isd.at[slot], isems[slot]).wait()
        for k in range(CH // 16):
            idst[slot, pl.ds(k * 16, 16)] = isd[slot, pl.ds(CH + k * 16, 16)]

    def issue_data(j, slot, b):
        g = base + j
        pltpu.async_copy(x_hbm.at[isd.at[slot, pl.ds(0, CH)]], rbufs[b],
                         gsems[b])
        pltpu.async_copy(e_hbm.at[pl.ds(g * CH, CH)], ebufs[b], esems[b])

    def wait_data(j, slot, b):
        g = base + j
        pltpu.make_async_copy(x_hbm.at[isd.at[slot, pl.ds(0, CH)]], rbufs[b],
                              gsems[b]).wait()
        pltpu.make_async_copy(e_hbm.at[pl.ds(g * CH, CH)], ebufs[b],
                              esems[b]).wait()

    def compute(b):
        rbuf = rbufs[b]
        ebuf = ebufs[b]

        def relu_row(r, inner):
            for k in range(D // 16):
                sl = pl.ds(k * 16, 16)
                rbuf[r, sl] = jnp.maximum(rbuf[r, sl] + ebuf[r, sl],
                                          jnp.float32(0.0))
            return inner

        lax.fori_loop(0, CH, relu_row, 0)

    def scatter(slot, b):
        pltpu.async_copy(rbufs[b], agg.at[idst.at[slot]], ssems[b],
                         add=True).wait()

    for sl in range(4):
        issue_idx(jnp.int32(sl), sl)
    wait_idx(jnp.int32(0), 0)
    issue_data(jnp.int32(0), 0, 0)

    def quad_body(i, carry):
        j = 4 * i
        wait_idx(j + 1, 1)
        issue_data(j + 1, 1, 1)
        wait_data(j, 0, 0)
        compute(0)
        scatter(0, 0)

        @pl.when(j + 4 < nj)
        def _():
            issue_idx(j + 4, 0)

        wait_idx(j + 2, 2)
        issue_data(j + 2, 2, 0)
        wait_data(j + 1, 1, 1)
        compute(1)
        scatter(1, 1)

        @pl.when(j + 5 < nj)
        def _():
            issue_idx(j + 5, 1)

        wait_idx(j + 3, 3)
        issue_data(j + 3, 3, 1)
        wait_data(j + 2, 2, 0)
        compute(0)
        scatter(2, 0)

        @pl.when(j + 6 < nj)
        def _():
            issue_idx(j + 6, 2)

        @pl.when(j + 4 < nj)
        def _():
            wait_idx(j + 4, 0)
            issue_data(j + 4, 0, 0)

        wait_data(j + 3, 3, 1)
        compute(1)
        scatter(3, 1)

        @pl.when(j + 7 < nj)
        def _():
            issue_idx(j + 7, 3)

        return carry

    lax.fori_loop(0, nj // 4, quad_body, 0)
    plsc.subcore_barrier()

    for k in range(ROWS_PER_TILE // CH):
        r = row0 + k * CH
        pltpu.sync_copy(agg.at[pl.ds(r, CH)], rows0.at[pl.ds(0, CH)])
        pltpu.sync_copy(rows0.at[pl.ds(0, CH)],
                        out_hbm.at[pl.ds(c * NP_ + r, CH)])


_EB = 2000


def _edge_mlp_body(ea, We1, be1, We2, be2, e1, e2):
    a = ea[...]
    e1[...] = jnp.dot(a, We1[...], preferred_element_type=jnp.float32) + be1[...]
    e2[...] = jnp.dot(a, We2[...], preferred_element_type=jnp.float32) + be2[...]


def _edge_mlp(ea, We1, be1, We2, be2):
    grid = (E // _EB,)
    return pl.pallas_call(
        _edge_mlp_body,
        grid=grid,
        in_specs=[
            pl.BlockSpec((_EB, ED), lambda i: (i, 0)),
            pl.BlockSpec((ED, D), lambda i: (0, 0)),
            pl.BlockSpec((1, D), lambda i: (0, 0)),
            pl.BlockSpec((ED, D), lambda i: (0, 0)),
            pl.BlockSpec((1, D), lambda i: (0, 0)),
        ],
        out_specs=[
            pl.BlockSpec((_EB, D), lambda i: (i, 0)),
            pl.BlockSpec((_EB, D), lambda i: (i, 0)),
        ],
        out_shape=[
            jax.ShapeDtypeStruct((EP, D), jnp.float32),
            jax.ShapeDtypeStruct((EP, D), jnp.float32),
        ],
    )(ea, We1, be1, We2, be2)


_NB = 1000


def _node_mlp1_body(x, a0, a1, W1a, b1a, W1b, b1b, h1):
    t = x[...] + a0[...] + a1[...]
    u = _leaky(jnp.dot(t, W1a[...], preferred_element_type=jnp.float32) + b1a[...])
    h = _leaky(jnp.dot(u, W1b[...], preferred_element_type=jnp.float32) + b1b[...])
    h1[...] = jnp.concatenate([h, jnp.zeros((_NB, 64), jnp.float32)], axis=1)


def _node_mlp1(x, a0, a1, W1a, b1a, W1b, b1b):
    grid = (N // _NB,)
    full_spec = pl.BlockSpec((_NB, D), lambda i: (i, 0))
    return pl.pallas_call(
        _node_mlp1_body,
        grid=grid,
        in_specs=[
            full_spec, full_spec, full_spec,
            pl.BlockSpec((D, 32), lambda i: (0, 0)),
            pl.BlockSpec((1, 32), lambda i: (0, 0)),
            pl.BlockSpec((32, 64), lambda i: (0, 0)),
            pl.BlockSpec((1, 64), lambda i: (0, 0)),
        ],
        out_specs=pl.BlockSpec((_NB, D), lambda i: (i, 0)),
        out_shape=jax.ShapeDtypeStruct((N, D), jnp.float32),
    )(x, a0, a1, W1a, b1a, W1b, b1b)


def _node_mlp2_pool_body(h1, a0, a1, W2a, b2a, W2b, b2b, batch, g):
    i = pl.program_id(0)
    t = h1[...][:, :64] + a0[...][:, :64] + a1[...][:, :64]
    u = _leaky(jnp.dot(t, W2a[...], preferred_element_type=jnp.float32) + b2a[...])
    h2 = _leaky(jnp.dot(u, W2b[...], preferred_element_type=jnp.float32) + b2b[...])
    onehot = (batch[...] == lax.broadcasted_iota(jnp.int32, (1, G), 1)
              ).astype(jnp.float32)
    contrib = lax.dot_general(onehot, h2, (((0,), (0,)), ((), ())),
                              preferred_element_type=jnp.float32)

    @pl.when(i == 0)
    def _():
        g[...] = contrib

    @pl.when(i > 0)
    def _():
        g[...] = g[...] + contrib


def _node_mlp2_pool(h1, a0, a1, W2a, b2a, W2b, b2b, batch2d):
    grid = (N // _NB,)
    full_spec = pl.BlockSpec((_NB, D), lambda i: (i, 0))
    return pl.pallas_call(
        _node_mlp2_pool_body,
        grid=grid,
        in_specs=[
            full_spec, full_spec, full_spec,
            pl.BlockSpec((64, 128), lambda i: (0, 0)),
            pl.BlockSpec((1, 128), lambda i: (0, 0)),
            pl.BlockSpec((128, 256), lambda i: (0, 0)),
            pl.BlockSpec((1, 256), lambda i: (0, 0)),
            pl.BlockSpec((_NB, 1), lambda i: (i, 0)),
        ],
        out_specs=pl.BlockSpec((G, 256), lambda i: (0, 0)),
        out_shape=jax.ShapeDtypeStruct((G, 256), jnp.float32),
    )(h1, a0, a1, W2a, b2a, W2b, b2b, batch2d)


def _head_body(g, Wf0, bf0, Wf1, bf1, Wf2, bf2, Wr, br, out):
    t = _leaky(jnp.dot(g[...], Wf0[...], preferred_element_type=jnp.float32) + bf0[...])
    t = _leaky(jnp.dot(t, Wf1[...], preferred_element_type=jnp.float32) + bf1[...])
    t = _leaky(jnp.dot(t, Wf2[...], preferred_element_type=jnp.float32) + bf2[...])
    out[...] = jnp.dot(t, Wr[...], preferred_element_type=jnp.float32) + br[...]


def _head(g, Wf0, bf0, Wf1, bf1, Wf2, bf2, Wr, br):
    return pl.pallas_call(
        _head_body,
        out_shape=jax.ShapeDtypeStruct((G, 1), jnp.float32),
    )(g, Wf0, bf0, Wf1, bf1, Wf2, bf2, Wr, br)


def kernel(x, edge_index, edge_attr, batch, We1, be1, W1a, b1a, W1b, b1b,
           We2, be2, W2a, b2a, W2b, b2b, Wf0, bf0, Wf1, bf1, Wf2, bf2, Wr, br):
    pad = EP - E
    srcf = jnp.concatenate([edge_index[0],
                            jnp.zeros((pad,), edge_index.dtype)])
    dstf = jnp.concatenate([edge_index[1],
                            jnp.full((pad,), NP_ - 1, edge_index.dtype)])
    sd = jnp.concatenate([srcf.reshape(NCH, CH),
                          dstf.reshape(NCH, CH)], axis=1).reshape(-1)
    batch2d = batch.reshape(N, 1)
    We2p = jnp.pad(We2, ((0, 0), (0, 64)))
    be2p = jnp.pad(be2, (0, 64)).reshape(1, D)

    e1, e2 = _edge_mlp(edge_attr, We1, be1.reshape(1, D), We2p, be2p)

    p1 = _sc_agg(x, sd, e1)
    h1p = _node_mlp1(x, p1[:N], p1[NP_:NP_ + N], W1a, b1a.reshape(1, 32),
                     W1b, b1b.reshape(1, 64))

    p2 = _sc_agg(h1p, sd, e2)
    g = _node_mlp2_pool(h1p, p2[:N], p2[NP_:NP_ + N], W2a, b2a.reshape(1, 128),
                        W2b, b2b.reshape(1, 256), batch2d)

    out = _head(g, Wf0, bf0.reshape(1, 128), Wf1, bf1.reshape(1, 64),
                Wf2, bf2.reshape(1, 32), Wr, br.reshape(1, 1))
    return out.reshape(G)


# speedup vs baseline: 1.0925x; 1.0035x over previous
import functools

import jax
import jax.numpy as jnp
from jax import lax
from jax.experimental import pallas as pl
from jax.experimental.pallas import tpu as pltpu
from jax.experimental.pallas import tpu_sc as plsc

N = 10000
E = 320000
D = 128
ED = 16
G = 64

NC = 2
NS = 16
NW = NC * NS
CH = 64
NCH = 5120
EP = NCH * CH
CN0 = 224
CN1 = 96
NP_ = 10240
ROWS_PER_TILE = NP_ // NS


def _leaky(v):
    return jnp.where(v > 0, v, v * jnp.float32(0.01))


_mesh = plsc.VectorSubcoreMesh(
    core_axis_name="c", subcore_axis_name="s", num_cores=NC, num_subcores=NS
)


@functools.partial(
    pl.kernel,
    out_type=jax.ShapeDtypeStruct((NC * NP_, D), jnp.float32),
    mesh=_mesh,
    scratch_types=[
        pltpu.VMEM((4, 2 * CH), jnp.int32),
        pltpu.VMEM((4, CH), jnp.int32),
        pltpu.VMEM((CH, D), jnp.float32),
        pltpu.VMEM((CH, D), jnp.float32),
        pltpu.VMEM((CH, D), jnp.float32),
        pltpu.VMEM((CH, D), jnp.float32),
        pltpu.VMEM_SHARED((NP_, D), jnp.float32),
        pltpu.SemaphoreType.DMA,
        pltpu.SemaphoreType.DMA,
        pltpu.SemaphoreType.DMA,
        pltpu.SemaphoreType.DMA,
        pltpu.SemaphoreType.DMA,
        pltpu.SemaphoreType.DMA,
        pltpu.SemaphoreType.DMA,
        pltpu.SemaphoreType.DMA,
        pltpu.SemaphoreType.DMA,
        pltpu.SemaphoreType.DMA,
    ],
)
def _sc_agg(x_hbm, sd_hbm, e_hbm, out_hbm, isd, idst,
            rows0, rows1, evs0, evs1, agg,
            si0, si1, si2, si3, sg0, sg1, se0, se1, ss0, ss1):
    c = lax.axis_index("c")
    s = lax.axis_index("s")
    nj = jnp.where(c == 0, CN0, CN1)
    base = jnp.where(c == 0, s * CN0, NS * CN0 + s * CN1)
    row0 = s * ROWS_PER_TILE

    isems = (si0, si1, si2, si3)
    rbufs = (rows0, rows1)
    ebufs = (evs0, evs1)
    gsems = (sg0, sg1)
    esems = (se0, se1)
    ssems = (ss0, ss1)

    def zero_row(i, carry):
        for k in range(D // 16):
            rows0[i, pl.ds(k * 16, 16)] = jnp.zeros((16,), jnp.float32)
        return carry

    lax.fori_loop(0, CH, zero_row, 0)
    for k in range(ROWS_PER_TILE // CH):
        pltpu.async_copy(rows0.at[pl.ds(0, CH)],
                         agg.at[pl.ds(row0 + k * CH, CH)], sg0)
    for k in range(ROWS_PER_TILE // CH):
        pltpu.make_async_copy(rows0.at[pl.ds(0, CH)],
                              agg.at[pl.ds(row0 + k * CH, CH)], sg0).wait()
    plsc.subcore_barrier()

    def issue_idx(j, slot):
        g = base + j
        pltpu.async_copy(sd_hbm.at[pl.ds(g * (2 * CH), 2 * CH)],
                         isd.at[slot], isems[slot])

    def wait_idx(j, slot):
        g = base + j
        pltpu.make_async_copy(sd_hbm.at[pl.ds(g * (2 * CH), 2 * CH)],
                              isd.at[slot], isems[slot]).wait()
        for k in range(CH // 16):
            idst[slot, pl.ds(k * 16, 16)] = isd[slot, pl.ds(CH + k * 16, 16)]

    def issue_data(j, slot, b):
        g = base + j
        pltpu.async_copy(x_hbm.at[isd.at[slot, pl.ds(0, CH)]], rbufs[b],
                         gsems[b])
        pltpu.async_copy(e_hbm.at[pl.ds(g * CH, CH)], ebufs[b], esems[b])

    def wait_data(j, slot, b):
        g = base + j
        pltpu.make_async_copy(x_hbm.at[isd.at[slot, pl.ds(0, CH)]], rbufs[b],
                              gsems[b]).wait()
        pltpu.make_async_copy(e_hbm.at[pl.ds(g * CH, CH)], ebufs[b],
                              esems[b]).wait()

    def compute(b):
        rbuf = rbufs[b]
        ebuf = ebufs[b]

        def relu_row(r, inner):
            for k in range(D // 16):
                sl = pl.ds(k * 16, 16)
                rbuf[r, sl] = jnp.maximum(rbuf[r, sl] + ebuf[r, sl],
                                          jnp.float32(0.0))
            return inner

        lax.fori_loop(0, CH, relu_row, 0)

    def scatter(slot, b):
        pltpu.async_copy(rbufs[b], agg.at[idst.at[slot]], ssems[b],
                         add=True).wait()

    for sl in range(4):
        issue_idx(jnp.int32(sl), sl)
    wait_idx(jnp.int32(0), 0)
    issue_data(jnp.int32(0), 0, 0)

    def quad_body(i, carry):
        j = 4 * i
        wait_idx(j + 1, 1)
        issue_data(j + 1, 1, 1)
        wait_data(j, 0, 0)
        compute(0)
        scatter(0, 0)

        @pl.when(j + 4 < nj)
        def _():
            issue_idx(j + 4, 0)

        wait_idx(j + 2, 2)
        issue_data(j + 2, 2, 0)
        wait_data(j + 1, 1, 1)
        compute(1)
        scatter(1, 1)

        @pl.when(j + 5 < nj)
        def _():
            issue_idx(j + 5, 1)

        wait_idx(j + 3, 3)
        issue_data(j + 3, 3, 1)
        wait_data(j + 2, 2, 0)
        compute(0)
        scatter(2, 0)

        @pl.when(j + 6 < nj)
        def _():
            issue_idx(j + 6, 2)

        @pl.when(j + 4 < nj)
        def _():
            wait_idx(j + 4, 0)
            issue_data(j + 4, 0, 0)

        wait_data(j + 3, 3, 1)
        compute(1)
        scatter(3, 1)

        @pl.when(j + 7 < nj)
        def _():
            issue_idx(j + 7, 3)

        return carry

    lax.fori_loop(0, nj // 4, quad_body, 0)
    plsc.subcore_barrier()

    nk = ROWS_PER_TILE // CH

    def sp2vm(k, b, issue):
        cp = (pltpu.async_copy if issue else pltpu.make_async_copy)
        return cp(agg.at[pl.ds(row0 + k * CH, CH)], rbufs[b], gsems[b])

    def vm2hbm(k, b, issue):
        cp = (pltpu.async_copy if issue else pltpu.make_async_copy)
        return cp(rbufs[b], out_hbm.at[pl.ds(c * NP_ + row0 + k * CH, CH)],
                  ssems[b])

    sp2vm(0, 0, True)
    for k in range(nk):
        b = k % 2
        sp2vm(k, b, False).wait()
        if k + 1 < nk:
            if k >= 1:
                vm2hbm(k - 1, 1 - b, False).wait()
            sp2vm(k + 1, 1 - b, True)
        vm2hbm(k, b, True)
    vm2hbm(nk - 2, nk % 2, False).wait()
    vm2hbm(nk - 1, (nk - 1) % 2, False).wait()


_EB = 2000


def _edge_mlp_body(ea, We1, be1, We2, be2, e1, e2):
    a = ea[...]
    e1[...] = jnp.dot(a, We1[...], preferred_element_type=jnp.float32) + be1[...]
    e2[...] = jnp.dot(a, We2[...], preferred_element_type=jnp.float32) + be2[...]


def _edge_mlp(ea, We1, be1, We2, be2):
    grid = (E // _EB,)
    return pl.pallas_call(
        _edge_mlp_body,
        grid=grid,
        in_specs=[
            pl.BlockSpec((_EB, ED), lambda i: (i, 0)),
            pl.BlockSpec((ED, D), lambda i: (0, 0)),
            pl.BlockSpec((1, D), lambda i: (0, 0)),
            pl.BlockSpec((ED, D), lambda i: (0, 0)),
            pl.BlockSpec((1, D), lambda i: (0, 0)),
        ],
        out_specs=[
            pl.BlockSpec((_EB, D), lambda i: (i, 0)),
            pl.BlockSpec((_EB, D), lambda i: (i, 0)),
        ],
        out_shape=[
            jax.ShapeDtypeStruct((EP, D), jnp.float32),
            jax.ShapeDtypeStruct((EP, D), jnp.float32),
        ],
    )(ea, We1, be1, We2, be2)


_NB = 1000


def _node_mlp1_body(x, a0, a1, W1a, b1a, W1b, b1b, h1):
    t = x[...] + a0[...] + a1[...]
    u = _leaky(jnp.dot(t, W1a[...], preferred_element_type=jnp.float32) + b1a[...])
    h = _leaky(jnp.dot(u, W1b[...], preferred_element_type=jnp.float32) + b1b[...])
    h1[...] = jnp.concatenate([h, jnp.zeros((_NB, 64), jnp.float32)], axis=1)


def _node_mlp1(x, a0, a1, W1a, b1a, W1b, b1b):
    grid = (N // _NB,)
    full_spec = pl.BlockSpec((_NB, D), lambda i: (i, 0))
    return pl.pallas_call(
        _node_mlp1_body,
        grid=grid,
        in_specs=[
            full_spec, full_spec, full_spec,
            pl.BlockSpec((D, 32), lambda i: (0, 0)),
            pl.BlockSpec((1, 32), lambda i: (0, 0)),
            pl.BlockSpec((32, 64), lambda i: (0, 0)),
            pl.BlockSpec((1, 64), lambda i: (0, 0)),
        ],
        out_specs=pl.BlockSpec((_NB, D), lambda i: (i, 0)),
        out_shape=jax.ShapeDtypeStruct((N, D), jnp.float32),
    )(x, a0, a1, W1a, b1a, W1b, b1b)


def _node_mlp2_pool_body(h1, a0, a1, W2a, b2a, W2b, b2b, batch, g):
    i = pl.program_id(0)
    t = h1[...][:, :64] + a0[...][:, :64] + a1[...][:, :64]
    u = _leaky(jnp.dot(t, W2a[...], preferred_element_type=jnp.float32) + b2a[...])
    h2 = _leaky(jnp.dot(u, W2b[...], preferred_element_type=jnp.float32) + b2b[...])
    onehot = (batch[...] == lax.broadcasted_iota(jnp.int32, (1, G), 1)
              ).astype(jnp.float32)
    contrib = lax.dot_general(onehot, h2, (((0,), (0,)), ((), ())),
                              preferred_element_type=jnp.float32)

    @pl.when(i == 0)
    def _():
        g[...] = contrib

    @pl.when(i > 0)
    def _():
        g[...] = g[...] + contrib


def _node_mlp2_pool(h1, a0, a1, W2a, b2a, W2b, b2b, batch2d):
    grid = (N // _NB,)
    full_spec = pl.BlockSpec((_NB, D), lambda i: (i, 0))
    return pl.pallas_call(
        _node_mlp2_pool_body,
        grid=grid,
        in_specs=[
            full_spec, full_spec, full_spec,
            pl.BlockSpec((64, 128), lambda i: (0, 0)),
            pl.BlockSpec((1, 128), lambda i: (0, 0)),
            pl.BlockSpec((128, 256), lambda i: (0, 0)),
            pl.BlockSpec((1, 256), lambda i: (0, 0)),
            pl.BlockSpec((_NB, 1), lambda i: (i, 0)),
        ],
        out_specs=pl.BlockSpec((G, 256), lambda i: (0, 0)),
        out_shape=jax.ShapeDtypeStruct((G, 256), jnp.float32),
    )(h1, a0, a1, W2a, b2a, W2b, b2b, batch2d)


def _head_body(g, Wf0, bf0, Wf1, bf1, Wf2, bf2, Wr, br, out):
    t = _leaky(jnp.dot(g[...], Wf0[...], preferred_element_type=jnp.float32) + bf0[...])
    t = _leaky(jnp.dot(t, Wf1[...], preferred_element_type=jnp.float32) + bf1[...])
    t = _leaky(jnp.dot(t, Wf2[...], preferred_element_type=jnp.float32) + bf2[...])
    out[...] = jnp.dot(t, Wr[...], preferred_element_type=jnp.float32) + br[...]


def _head(g, Wf0, bf0, Wf1, bf1, Wf2, bf2, Wr, br):
    return pl.pallas_call(
        _head_body,
        out_shape=jax.ShapeDtypeStruct((G, 1), jnp.float32),
    )(g, Wf0, bf0, Wf1, bf1, Wf2, bf2, Wr, br)


def kernel(x, edge_index, edge_attr, batch, We1, be1, W1a, b1a, W1b, b1b,
           We2, be2, W2a, b2a, W2b, b2b, Wf0, bf0, Wf1, bf1, Wf2, bf2, Wr, br):
    pad = EP - E
    srcf = jnp.concatenate([edge_index[0],
                            jnp.zeros((pad,), edge_index.dtype)])
    dstf = jnp.concatenate([edge_index[1],
                            jnp.full((pad,), NP_ - 1, edge_index.dtype)])
    sd = jnp.concatenate([srcf.reshape(NCH, CH),
                          dstf.reshape(NCH, CH)], axis=1).reshape(-1)
    batch2d = batch.reshape(N, 1)
    We2p = jnp.pad(We2, ((0, 0), (0, 64)))
    be2p = jnp.pad(be2, (0, 64)).reshape(1, D)

    e1, e2 = _edge_mlp(edge_attr, We1, be1.reshape(1, D), We2p, be2p)

    p1 = _sc_agg(x, sd, e1)
    h1p = _node_mlp1(x, p1[:N], p1[NP_:NP_ + N], W1a, b1a.reshape(1, 32),
                     W1b, b1b.reshape(1, 64))

    p2 = _sc_agg(h1p, sd, e2)
    g = _node_mlp2_pool(h1p, p2[:N], p2[NP_:NP_ + N], W2a, b2a.reshape(1, 128),
                        W2b, b2b.reshape(1, 256), batch2d)

    out = _head(g, Wf0, bf0.reshape(1, 128), Wf1, bf1.reshape(1, 64),
                Wf2, bf2.reshape(1, 32), Wr, br.reshape(1, 1))
    return out.reshape(G)


# speedup vs baseline: 1.1218x; 1.0268x over previous
import functools

import jax
import jax.numpy as jnp
from jax import lax
from jax.experimental import pallas as pl
from jax.experimental.pallas import tpu as pltpu
from jax.experimental.pallas import tpu_sc as plsc

N = 10000
E = 320000
D = 128
ED = 16
G = 64

NC = 2
NS = 16
NW = NC * NS
CH = 64
NCH = 5120
EP = NCH * CH
CN0 = 296
CN1 = 24
NP_ = 10240
ROWS_PER_TILE = NP_ // NS


def _leaky(v):
    return jnp.where(v > 0, v, v * jnp.float32(0.01))


_mesh = plsc.VectorSubcoreMesh(
    core_axis_name="c", subcore_axis_name="s", num_cores=NC, num_subcores=NS
)


@functools.partial(
    pl.kernel,
    out_type=jax.ShapeDtypeStruct((NC * NP_, D), jnp.float32),
    mesh=_mesh,
    scratch_types=[
        pltpu.VMEM((4, 2 * CH), jnp.int32),
        pltpu.VMEM((4, CH), jnp.int32),
        pltpu.VMEM((CH, D), jnp.float32),
        pltpu.VMEM((CH, D), jnp.float32),
        pltpu.VMEM((CH, D), jnp.float32),
        pltpu.VMEM((CH, D), jnp.float32),
        pltpu.VMEM_SHARED((NP_, D), jnp.float32),
        pltpu.SemaphoreType.DMA,
        pltpu.SemaphoreType.DMA,
        pltpu.SemaphoreType.DMA,
        pltpu.SemaphoreType.DMA,
        pltpu.SemaphoreType.DMA,
        pltpu.SemaphoreType.DMA,
        pltpu.SemaphoreType.DMA,
        pltpu.SemaphoreType.DMA,
        pltpu.SemaphoreType.DMA,
        pltpu.SemaphoreType.DMA,
    ],
)
def _sc_agg(x_hbm, sd_hbm, e_hbm, out_hbm, isd, idst,
            rows0, rows1, evs0, evs1, agg,
            si0, si1, si2, si3, sg0, sg1, se0, se1, ss0, ss1):
    c = lax.axis_index("c")
    s = lax.axis_index("s")
    nj = jnp.where(c == 0, CN0, CN1)
    base = jnp.where(c == 0, s * CN0, NS * CN0 + s * CN1)
    row0 = s * ROWS_PER_TILE

    isems = (si0, si1, si2, si3)
    rbufs = (rows0, rows1)
    ebufs = (evs0, evs1)
    gsems = (sg0, sg1)
    esems = (se0, se1)
    ssems = (ss0, ss1)

    def zero_row(i, carry):
        for k in range(D // 16):
            rows0[i, pl.ds(k * 16, 16)] = jnp.zeros((16,), jnp.float32)
        return carry

    lax.fori_loop(0, CH, zero_row, 0)
    for k in range(ROWS_PER_TILE // CH):
        pltpu.async_copy(rows0.at[pl.ds(0, CH)],
                         agg.at[pl.ds(row0 + k * CH, CH)], sg0)
    for k in range(ROWS_PER_TILE // CH):
        pltpu.make_async_copy(rows0.at[pl.ds(0, CH)],
                              agg.at[pl.ds(row0 + k * CH, CH)], sg0).wait()
    plsc.subcore_barrier()

    def issue_idx(j, slot):
        g = base + j
        pltpu.async_copy(sd_hbm.at[pl.ds(g * (2 * CH), 2 * CH)],
                         isd.at[slot], isems[slot])

    def wait_idx(j, slot):
        g = base + j
        pltpu.make_async_copy(sd_hbm.at[pl.ds(g * (2 * CH), 2 * CH)],
                              isd.at[slot], isems[slot]).wait()
        for k in range(CH // 16):
            idst[slot, pl.ds(k * 16, 16)] = isd[slot, pl.ds(CH + k * 16, 16)]

    def issue_data(j, slot, b):
        g = base + j
        pltpu.async_copy(x_hbm.at[isd.at[slot, pl.ds(0, CH)]], rbufs[b],
                         gsems[b])
        pltpu.async_copy(e_hbm.at[pl.ds(g * CH, CH)], ebufs[b], esems[b])

    def wait_data(j, slot, b):
        g = base + j
        pltpu.make_async_copy(x_hbm.at[isd.at[slot, pl.ds(0, CH)]], rbufs[b],
                              gsems[b]).wait()
        pltpu.make_async_copy(e_hbm.at[pl.ds(g * CH, CH)], ebufs[b],
                              esems[b]).wait()

    def compute(b):
        rbuf = rbufs[b]
        ebuf = ebufs[b]

        def relu_row(r, inner):
            for k in range(D // 16):
                sl = pl.ds(k * 16, 16)
                rbuf[r, sl] = jnp.maximum(rbuf[r, sl] + ebuf[r, sl],
                                          jnp.float32(0.0))
            return inner

        lax.fori_loop(0, CH, relu_row, 0)

    def scatter(slot, b):
        pltpu.async_copy(rbufs[b], agg.at[idst.at[slot]], ssems[b],
                         add=True).wait()

    for sl in range(4):
        issue_idx(jnp.int32(sl), sl)
    wait_idx(jnp.int32(0), 0)
    issue_data(jnp.int32(0), 0, 0)

    def quad_body(i, carry):
        j = 4 * i
        wait_idx(j + 1, 1)
        issue_data(j + 1, 1, 1)
        wait_data(j, 0, 0)
        compute(0)
        scatter(0, 0)

        @pl.when(j + 4 < nj)
        def _():
            issue_idx(j + 4, 0)

        wait_idx(j + 2, 2)
        issue_data(j + 2, 2, 0)
        wait_data(j + 1, 1, 1)
        compute(1)
        scatter(1, 1)

        @pl.when(j + 5 < nj)
        def _():
            issue_idx(j + 5, 1)

        wait_idx(j + 3, 3)
        issue_data(j + 3, 3, 1)
        wait_data(j + 2, 2, 0)
        compute(0)
        scatter(2, 0)

        @pl.when(j + 6 < nj)
        def _():
            issue_idx(j + 6, 2)

        @pl.when(j + 4 < nj)
        def _():
            wait_idx(j + 4, 0)
            issue_data(j + 4, 0, 0)

        wait_data(j + 3, 3, 1)
        compute(1)
        scatter(3, 1)

        @pl.when(j + 7 < nj)
        def _():
            issue_idx(j + 7, 3)

        return carry

    lax.fori_loop(0, nj // 4, quad_body, 0)
    plsc.subcore_barrier()

    nk = ROWS_PER_TILE // CH

    def sp2vm(k, b, issue):
        cp = (pltpu.async_copy if issue else pltpu.make_async_copy)
        return cp(agg.at[pl.ds(row0 + k * CH, CH)], rbufs[b], gsems[b])

    def vm2hbm(k, b, issue):
        cp = (pltpu.async_copy if issue else pltpu.make_async_copy)
        return cp(rbufs[b], out_hbm.at[pl.ds(c * NP_ + row0 + k * CH, CH)],
                  ssems[b])

    sp2vm(0, 0, True)
    for k in range(nk):
        b = k % 2
        sp2vm(k, b, False).wait()
        if k + 1 < nk:
            if k >= 1:
                vm2hbm(k - 1, 1 - b, False).wait()
            sp2vm(k + 1, 1 - b, True)
        vm2hbm(k, b, True)
    vm2hbm(nk - 2, nk % 2, False).wait()
    vm2hbm(nk - 1, (nk - 1) % 2, False).wait()


_EB = 2000


def _edge_mlp_body(ea, We1, be1, We2, be2, e1, e2):
    a = ea[...]
    e1[...] = jnp.dot(a, We1[...], preferred_element_type=jnp.float32) + be1[...]
    e2[...] = jnp.dot(a, We2[...], preferred_element_type=jnp.float32) + be2[...]


def _edge_mlp(ea, We1, be1, We2, be2):
    grid = (E // _EB,)
    return pl.pallas_call(
        _edge_mlp_body,
        grid=grid,
        in_specs=[
            pl.BlockSpec((_EB, ED), lambda i: (i, 0)),
            pl.BlockSpec((ED, D), lambda i: (0, 0)),
            pl.BlockSpec((1, D), lambda i: (0, 0)),
            pl.BlockSpec((ED, D), lambda i: (0, 0)),
            pl.BlockSpec((1, D), lambda i: (0, 0)),
        ],
        out_specs=[
            pl.BlockSpec((_EB, D), lambda i: (i, 0)),
            pl.BlockSpec((_EB, D), lambda i: (i, 0)),
        ],
        out_shape=[
            jax.ShapeDtypeStruct((EP, D), jnp.float32),
            jax.ShapeDtypeStruct((EP, D), jnp.float32),
        ],
    )(ea, We1, be1, We2, be2)


_NB = 1000


def _node_mlp1_body(x, a0, a1, W1a, b1a, W1b, b1b, h1):
    t = x[...] + a0[...] + a1[...]
    u = _leaky(jnp.dot(t, W1a[...], preferred_element_type=jnp.float32) + b1a[...])
    h = _leaky(jnp.dot(u, W1b[...], preferred_element_type=jnp.float32) + b1b[...])
    h1[...] = jnp.concatenate([h, jnp.zeros((_NB, 64), jnp.float32)], axis=1)


def _node_mlp1(x, a0, a1, W1a, b1a, W1b, b1b):
    grid = (N // _NB,)
    full_spec = pl.BlockSpec((_NB, D), lambda i: (i, 0))
    return pl.pallas_call(
        _node_mlp1_body,
        grid=grid,
        in_specs=[
            full_spec, full_spec, full_spec,
            pl.BlockSpec((D, 32), lambda i: (0, 0)),
            pl.BlockSpec((1, 32), lambda i: (0, 0)),
            pl.BlockSpec((32, 64), lambda i: (0, 0)),
            pl.BlockSpec((1, 64), lambda i: (0, 0)),
        ],
        out_specs=pl.BlockSpec((_NB, D), lambda i: (i, 0)),
        out_shape=jax.ShapeDtypeStruct((N, D), jnp.float32),
    )(x, a0, a1, W1a, b1a, W1b, b1b)


def _node_mlp2_pool_body(h1, a0, a1, W2a, b2a, W2b, b2b, batch, g):
    i = pl.program_id(0)
    t = h1[...][:, :64] + a0[...][:, :64] + a1[...][:, :64]
    u = _leaky(jnp.dot(t, W2a[...], preferred_element_type=jnp.float32) + b2a[...])
    h2 = _leaky(jnp.dot(u, W2b[...], preferred_element_type=jnp.float32) + b2b[...])
    onehot = (batch[...] == lax.broadcasted_iota(jnp.int32, (1, G), 1)
              ).astype(jnp.float32)
    contrib = lax.dot_general(onehot, h2, (((0,), (0,)), ((), ())),
                              preferred_element_type=jnp.float32)

    @pl.when(i == 0)
    def _():
        g[...] = contrib

    @pl.when(i > 0)
    def _():
        g[...] = g[...] + contrib


def _node_mlp2_pool(h1, a0, a1, W2a, b2a, W2b, b2b, batch2d):
    grid = (N // _NB,)
    full_spec = pl.BlockSpec((_NB, D), lambda i: (i, 0))
    return pl.pallas_call(
        _node_mlp2_pool_body,
        grid=grid,
        in_specs=[
            full_spec, full_spec, full_spec,
            pl.BlockSpec((64, 128), lambda i: (0, 0)),
            pl.BlockSpec((1, 128), lambda i: (0, 0)),
            pl.BlockSpec((128, 256), lambda i: (0, 0)),
            pl.BlockSpec((1, 256), lambda i: (0, 0)),
            pl.BlockSpec((_NB, 1), lambda i: (i, 0)),
        ],
        out_specs=pl.BlockSpec((G, 256), lambda i: (0, 0)),
        out_shape=jax.ShapeDtypeStruct((G, 256), jnp.float32),
    )(h1, a0, a1, W2a, b2a, W2b, b2b, batch2d)


def _head_body(g, Wf0, bf0, Wf1, bf1, Wf2, bf2, Wr, br, out):
    t = _leaky(jnp.dot(g[...], Wf0[...], preferred_element_type=jnp.float32) + bf0[...])
    t = _leaky(jnp.dot(t, Wf1[...], preferred_element_type=jnp.float32) + bf1[...])
    t = _leaky(jnp.dot(t, Wf2[...], preferred_element_type=jnp.float32) + bf2[...])
    out[...] = jnp.dot(t, Wr[...], preferred_element_type=jnp.float32) + br[...]


def _head(g, Wf0, bf0, Wf1, bf1, Wf2, bf2, Wr, br):
    return pl.pallas_call(
        _head_body,
        out_shape=jax.ShapeDtypeStruct((G, 1), jnp.float32),
    )(g, Wf0, bf0, Wf1, bf1, Wf2, bf2, Wr, br)


def kernel(x, edge_index, edge_attr, batch, We1, be1, W1a, b1a, W1b, b1b,
           We2, be2, W2a, b2a, W2b, b2b, Wf0, bf0, Wf1, bf1, Wf2, bf2, Wr, br):
    pad = EP - E
    srcf = jnp.concatenate([edge_index[0],
                            jnp.zeros((pad,), edge_index.dtype)])
    dstf = jnp.concatenate([edge_index[1],
                            jnp.full((pad,), NP_ - 1, edge_index.dtype)])
    sd = jnp.concatenate([srcf.reshape(NCH, CH),
                          dstf.reshape(NCH, CH)], axis=1).reshape(-1)
    batch2d = batch.reshape(N, 1)
    We2p = jnp.pad(We2, ((0, 0), (0, 64)))
    be2p = jnp.pad(be2, (0, 64)).reshape(1, D)

    e1, e2 = _edge_mlp(edge_attr, We1, be1.reshape(1, D), We2p, be2p)

    p1 = _sc_agg(x, sd, e1)
    h1p = _node_mlp1(x, p1[:N], p1[NP_:NP_ + N], W1a, b1a.reshape(1, 32),
                     W1b, b1b.reshape(1, 64))

    p2 = _sc_agg(h1p, sd, e2)
    g = _node_mlp2_pool(h1p, p2[:N], p2[NP_:NP_ + N], W2a, b2a.reshape(1, 128),
                        W2b, b2b.reshape(1, 256), batch2d)

    out = _head(g, Wf0, bf0.reshape(1, 128), Wf1, bf1.reshape(1, 64),
                Wf2, bf2.reshape(1, 32), Wr, br.reshape(1, 1))
    return out.reshape(G)
